# baseline (speedup 1.0000x reference)
import functools

import jax
import jax.numpy as jnp
from jax import lax
from jax.experimental import pallas as pl
from jax.experimental.pallas import tpu as pltpu
from jax.experimental.pallas import tpu_sc as plsc

B, N, K = 64, 32768, 512
NC, NS, L = 2, 16, 16
NW = NC * NS
RW = B // NW
NEG = -1000000000.0

_mesh = plsc.VectorSubcoreMesh(core_axis_name="c", subcore_axis_name="s")


@functools.partial(
    pl.kernel,
    mesh=_mesh,
    out_type=jax.ShapeDtypeStruct((B, N), jnp.float32),
    scratch_types=[
        pltpu.VMEM((RW, N), jnp.float32),
        pltpu.VMEM((1, N), jnp.float32),
        pltpu.VMEM((RW, K), jnp.int32),
        pltpu.SemaphoreType.DMA,
        pltpu.SemaphoreType.DMA,
        pltpu.SemaphoreType.DMA,
        pltpu.SemaphoreType.DMA,
    ],
    compiler_params=pltpu.CompilerParams(needs_layout_passes=False),
)
def _filter_moves(x_hbm, mv_hbm, out_hbm, obuf, xrow, idx,
                  semi, semx, semo0, semo1):
    wid = lax.axis_index("s") * NC + lax.axis_index("c")
    row0 = wid * RW
    neg = jnp.full((L,), NEG, jnp.float32)
    z = jnp.full((L,), 0, jnp.int32)
    semo = [semo0, semo1]

    icopy = pltpu.async_copy(mv_hbm.at[pl.ds(row0, RW)], idx, semi)
    xcopy = pltpu.async_copy(x_hbm.at[row0], xrow.at[0], semx)

    def make_fill(r):
        def fill(i, _):
            base = i * (8 * L)
            for j in range(8):
                obuf[r, pl.ds(base + j * L, L)] = neg
            return 0
        return fill

    ocopies = []
    for r in range(RW):
        rv = jnp.full((L,), r, jnp.int32)
        lax.fori_loop(0, N // (8 * L), make_fill(r), 0)
        if r == 0:
            icopy.wait()
        xcopy.wait()

        def scat(c, _):
            iv = idx[r, pl.ds(c * L, L)]
            v = plsc.load_gather(xrow, [z, iv])
            v = jnp.where(v == 0.0, jnp.float32(NEG), v)
            plsc.store_scatter(obuf, [rv, iv], v)
            return 0

        lax.fori_loop(0, K // L, scat, 0)
        ocopies.append(
            pltpu.async_copy(obuf.at[r], out_hbm.at[row0 + r], semo[r]))
        if r + 1 < RW:
            xcopy = pltpu.async_copy(x_hbm.at[row0 + r + 1], xrow.at[0], semx)
    for cp in ocopies:
        cp.wait()


def kernel(x, possible_moves):
    return _filter_moves(x, possible_moves.astype(jnp.int32))
